# baseline (speedup 1.0000x reference)
import functools

import jax
import jax.numpy as jnp
from jax import lax
from jax.experimental import pallas as pl
from jax.experimental.pallas import tpu as pltpu
from jax.experimental.pallas import tpu_sc as plsc

NC = 2
NS = 16
NW = NC * NS
CH = 96


NBUF = 4


@functools.lru_cache(maxsize=None)
def _make_sc_scatter(n_pad: int, e_pad: int, d: int):
    ew = e_pad // NW
    nch = ew // CH
    rps = n_pad // NS
    mesh = plsc.VectorSubcoreMesh(core_axis_name="c", subcore_axis_name="s")

    @functools.partial(
        pl.kernel,
        out_type=jax.ShapeDtypeStruct((NC * n_pad, d), jnp.float32),
        mesh=mesh,
        scratch_types=(
            [pltpu.VMEM_SHARED((n_pad, d), jnp.float32)]
            + [pltpu.VMEM((2, CH), jnp.int32) for _ in range(NBUF)]
            + [pltpu.VMEM((CH, d), jnp.float32) for _ in range(NBUF)]
            + [pltpu.SemaphoreType.DMA for _ in range(2 * NBUF)]
        ),
    )
    def sc_scatter(h_hbm, ed_hbm, zeros_hbm, out_hbm, acc, *bufs):
        idx = bufs[0:NBUF]
        rows = bufs[NBUF:2 * NBUF]
        gsem = bufs[2 * NBUF:3 * NBUF]
        ssem = bufs[3 * NBUF:4 * NBUF]
        c = lax.axis_index("c")
        s = lax.axis_index("s")
        wid = c * NS + s
        pltpu.sync_copy(zeros_hbm.at[pl.ds(s * rps, rps)],
                        acc.at[pl.ds(s * rps, rps)])
        plsc.subcore_barrier()

        cbase = wid * nch

        def start_gather(g, b):
            pltpu.sync_copy(ed_hbm.at[cbase + g], idx[b])
            pltpu.async_copy(h_hbm.at[idx[b].at[0]], rows[b], gsem[b])

        for b in range(NBUF):
            start_gather(b, b)

        def body(i, carry):
            a = i * NBUF
            for b in range(NBUF):
                pltpu.make_async_copy(h_hbm.at[idx[b].at[0]], rows[b],
                                      gsem[b]).wait()
                pltpu.async_copy(rows[b], acc.at[idx[b].at[1]], ssem[b],
                                 add=True)
            for b in range(NBUF):
                pltpu.make_async_copy(rows[b], acc.at[idx[b].at[1]],
                                      ssem[b]).wait()
                start_gather(a + NBUF + b, b)
            return carry

        lax.fori_loop(0, nch // NBUF, body, 0)
        for b in range(NBUF):
            pltpu.make_async_copy(h_hbm.at[idx[b].at[0]], rows[b],
                                  gsem[b]).wait()

        plsc.subcore_barrier()
        pltpu.sync_copy(acc.at[pl.ds(s * rps, rps)],
                        out_hbm.at[pl.ds(c * n_pad + s * rps, rps)])

    return sc_scatter


def _tc_layer_body(x_ref, p0_ref, p1_ref, w1_ref, b1_ref, w2_ref, b2_ref,
                   g_ref, be_ref, eps_ref, out_ref):
    h = (1.0 + eps_ref[0, 0]) * x_ref[...] + p0_ref[...] + p1_ref[...]
    h = jnp.dot(h, w1_ref[...], preferred_element_type=jnp.float32) + b1_ref[...]
    h = jnp.maximum(h, 0.0)
    h = jnp.dot(h, w2_ref[...], preferred_element_type=jnp.float32) + b2_ref[...]
    h = jnp.maximum(h, 0.0)
    mu = jnp.mean(h, axis=0, keepdims=True)
    var = jnp.mean((h - mu) ** 2, axis=0, keepdims=True)
    out_ref[...] = (g_ref[...] * (h - mu) * lax.rsqrt(var + 1e-5)
                    + be_ref[...])


def _tc_layer(x, p0, p1, p):
    n, d = x.shape
    h = p["W1"].shape[1]
    return pl.pallas_call(
        _tc_layer_body,
        out_shape=jax.ShapeDtypeStruct((n, h), jnp.float32),
    )(x, p0, p1, p["W1"], p["b1"].reshape(1, h), p["W2"],
      p["b2"].reshape(1, h), p["gamma"].reshape(1, h),
      p["beta"].reshape(1, h), p["eps"].reshape(1, 1))


def _tc_head_body(h1_ref, h2_ref, h3_ref, batch_ref, w1_ref, b1_ref,
                  w2_ref, b2_ref, out_ref, *, g: int):
    b = batch_ref[...]
    gid = lax.broadcasted_iota(jnp.int32, (g, b.shape[1]), 0)
    onehot = jnp.where(b == gid, 1.0, 0.0)
    counts = jnp.maximum(jnp.sum(onehot, axis=1, keepdims=True), 1.0)
    s1 = jnp.dot(onehot, h1_ref[...], preferred_element_type=jnp.float32)
    s2 = jnp.dot(onehot, h2_ref[...], preferred_element_type=jnp.float32)
    s3 = jnp.dot(onehot, h3_ref[...], preferred_element_type=jnp.float32)
    pooled = jnp.concatenate([s1, s2, s3], axis=1) / counts
    z = jnp.dot(pooled, w1_ref[...], preferred_element_type=jnp.float32)
    z = jnp.maximum(z + b1_ref[...], 0.0)
    logits = jnp.dot(z, w2_ref[...], preferred_element_type=jnp.float32)
    logits = logits + b2_ref[...]
    m = jnp.max(logits, axis=1, keepdims=True)
    shifted = logits - m
    out_ref[...] = shifted - jnp.log(
        jnp.sum(jnp.exp(shifted), axis=1, keepdims=True))


def _tc_head(h1, h2, h3, batch, params):
    g = 128
    c = params["fc2_W"].shape[1]
    n = h1.shape[0]
    hdim = params["fc1_W"].shape[1]
    return pl.pallas_call(
        functools.partial(_tc_head_body, g=g),
        out_shape=jax.ShapeDtypeStruct((g, c), jnp.float32),
    )(h1, h2, h3, batch.reshape(1, n).astype(jnp.int32),
      params["fc1_W"], params["fc1_b"].reshape(1, hdim),
      params["fc2_W"], params["fc2_b"].reshape(1, c))


def kernel(x, edge_index, batch, params):
    n, d = x.shape
    e = edge_index.shape[1]
    n_pad = ((n + NS * 8 - 1) // (NS * 8)) * NS * 8
    e_pad = ((e + NW * CH * NBUF - 1) // (NW * CH * NBUF)) * NW * CH * NBUF
    src = edge_index[0].astype(jnp.int32)
    dst = edge_index[1].astype(jnp.int32)
    pad = e_pad + NBUF * CH - e
    pad_ar = jnp.arange(pad, dtype=jnp.int32)
    src = jnp.concatenate([src, (pad_ar * 97) % n])
    dst = jnp.concatenate([dst, n + pad_ar % (n_pad - n)])
    ed = jnp.stack([src.reshape(-1, CH), dst.reshape(-1, CH)], axis=1)
    zeros = jnp.zeros((n_pad, d), jnp.float32)

    sc_scatter = _make_sc_scatter(n_pad, e_pad, d)

    hs = []
    h = x
    for p in params["layers"]:
        parts = sc_scatter(h, ed, zeros)
        p0 = parts[0:n]
        p1 = parts[n_pad:n_pad + n]
        h = _tc_layer(h, p0, p1, p)
        hs.append(h)

    return _tc_head(hs[0], hs[1], hs[2], batch, params)


# speedup vs baseline: 1.0530x; 1.0530x over previous
import functools

import jax
import jax.numpy as jnp
from jax import lax
from jax.experimental import pallas as pl
from jax.experimental.pallas import tpu as pltpu
from jax.experimental.pallas import tpu_sc as plsc

NC = 2
NS = 16
NW = NC * NS
CH = 128


NBUF = 3


@functools.lru_cache(maxsize=None)
def _make_sc_scatter(n_pad: int, e_pad: int, d: int):
    ew = e_pad // NW
    nch = ew // CH
    rps = n_pad // NS
    mesh = plsc.VectorSubcoreMesh(core_axis_name="c", subcore_axis_name="s")

    @functools.partial(
        pl.kernel,
        out_type=jax.ShapeDtypeStruct((NC * n_pad, d), jnp.float32),
        mesh=mesh,
        scratch_types=(
            [pltpu.VMEM_SHARED((n_pad, d), jnp.float32)]
            + [pltpu.VMEM((2, CH), jnp.int32) for _ in range(NBUF)]
            + [pltpu.VMEM((CH, d), jnp.float32) for _ in range(NBUF)]
            + [pltpu.SemaphoreType.DMA for _ in range(2 * NBUF)]
        ),
    )
    def sc_scatter(h_hbm, ed_hbm, zeros_hbm, out_hbm, acc, *bufs):
        idx = bufs[0:NBUF]
        rows = bufs[NBUF:2 * NBUF]
        gsem = bufs[2 * NBUF:3 * NBUF]
        ssem = bufs[3 * NBUF:4 * NBUF]
        c = lax.axis_index("c")
        s = lax.axis_index("s")
        wid = c * NS + s
        pltpu.sync_copy(zeros_hbm.at[pl.ds(s * rps, rps)],
                        acc.at[pl.ds(s * rps, rps)])
        plsc.subcore_barrier()

        cbase = wid * nch

        def start_gather(g, b):
            pltpu.sync_copy(ed_hbm.at[cbase + g], idx[b])
            pltpu.async_copy(h_hbm.at[idx[b].at[0]], rows[b], gsem[b])

        for b in range(NBUF):
            start_gather(b, b)

        def body(i, carry):
            a = i * NBUF
            for b in range(NBUF):
                pltpu.make_async_copy(h_hbm.at[idx[b].at[0]], rows[b],
                                      gsem[b]).wait()
                pltpu.async_copy(rows[b], acc.at[idx[b].at[1]], ssem[b],
                                 add=True)
            for b in range(NBUF):
                pltpu.make_async_copy(rows[b], acc.at[idx[b].at[1]],
                                      ssem[b]).wait()
                start_gather(a + NBUF + b, b)
            return carry

        lax.fori_loop(0, nch // NBUF, body, 0)
        for b in range(NBUF):
            pltpu.make_async_copy(h_hbm.at[idx[b].at[0]], rows[b],
                                  gsem[b]).wait()

        plsc.subcore_barrier()
        pltpu.sync_copy(acc.at[pl.ds(s * rps, rps)],
                        out_hbm.at[pl.ds(c * n_pad + s * rps, rps)])

    return sc_scatter


def _tc_layer_body(x_ref, parts_ref, w1_ref, b1_ref, w2_ref, b2_ref,
                   g_ref, be_ref, eps_ref, out_ref, *, n_pad: int):
    n = x_ref.shape[0]
    h = ((1.0 + eps_ref[0, 0]) * x_ref[...]
         + parts_ref[0:n] + parts_ref[n_pad:n_pad + n])
    h = jnp.dot(h, w1_ref[...], preferred_element_type=jnp.float32) + b1_ref[...]
    h = jnp.maximum(h, 0.0)
    h = jnp.dot(h, w2_ref[...], preferred_element_type=jnp.float32) + b2_ref[...]
    h = jnp.maximum(h, 0.0)
    mu = jnp.mean(h, axis=0, keepdims=True)
    var = jnp.mean((h - mu) ** 2, axis=0, keepdims=True)
    out_ref[...] = (g_ref[...] * (h - mu) * lax.rsqrt(var + 1e-5)
                    + be_ref[...])


def _tc_layer(x, parts, n_pad, p):
    n, d = x.shape
    h = p["W1"].shape[1]
    return pl.pallas_call(
        functools.partial(_tc_layer_body, n_pad=n_pad),
        out_shape=jax.ShapeDtypeStruct((n, h), jnp.float32),
    )(x, parts, p["W1"], p["b1"].reshape(1, h), p["W2"],
      p["b2"].reshape(1, h), p["gamma"].reshape(1, h),
      p["beta"].reshape(1, h), p["eps"].reshape(1, 1))


def _tc_head_body(h1_ref, h2_ref, h3_ref, batch_ref, w1_ref, b1_ref,
                  w2_ref, b2_ref, out_ref, *, g: int):
    b = batch_ref[...]
    gid = lax.broadcasted_iota(jnp.int32, (g, b.shape[1]), 0)
    onehot = jnp.where(b == gid, 1.0, 0.0)
    counts = jnp.maximum(jnp.sum(onehot, axis=1, keepdims=True), 1.0)
    s1 = jnp.dot(onehot, h1_ref[...], preferred_element_type=jnp.float32)
    s2 = jnp.dot(onehot, h2_ref[...], preferred_element_type=jnp.float32)
    s3 = jnp.dot(onehot, h3_ref[...], preferred_element_type=jnp.float32)
    pooled = jnp.concatenate([s1, s2, s3], axis=1) / counts
    z = jnp.dot(pooled, w1_ref[...], preferred_element_type=jnp.float32)
    z = jnp.maximum(z + b1_ref[...], 0.0)
    logits = jnp.dot(z, w2_ref[...], preferred_element_type=jnp.float32)
    logits = logits + b2_ref[...]
    m = jnp.max(logits, axis=1, keepdims=True)
    shifted = logits - m
    out_ref[...] = shifted - jnp.log(
        jnp.sum(jnp.exp(shifted), axis=1, keepdims=True))


def _tc_head(h1, h2, h3, batch, params):
    g = 128
    c = params["fc2_W"].shape[1]
    n = h1.shape[0]
    hdim = params["fc1_W"].shape[1]
    return pl.pallas_call(
        functools.partial(_tc_head_body, g=g),
        out_shape=jax.ShapeDtypeStruct((g, c), jnp.float32),
    )(h1, h2, h3, batch.reshape(1, n).astype(jnp.int32),
      params["fc1_W"], params["fc1_b"].reshape(1, hdim),
      params["fc2_W"], params["fc2_b"].reshape(1, c))


def kernel(x, edge_index, batch, params):
    n, d = x.shape
    e = edge_index.shape[1]
    n_pad = ((n + NS * 8 - 1) // (NS * 8)) * NS * 8
    e_pad = ((e + NW * CH * NBUF - 1) // (NW * CH * NBUF)) * NW * CH * NBUF
    src = edge_index[0].astype(jnp.int32)
    dst = edge_index[1].astype(jnp.int32)
    pad = e_pad + NBUF * CH - e
    pad_ar = jnp.arange(pad, dtype=jnp.int32)
    src = jnp.concatenate([src, (pad_ar * 97) % n])
    dst = jnp.concatenate([dst, n + pad_ar % (n_pad - n)])
    ed = jnp.stack([src.reshape(-1, CH), dst.reshape(-1, CH)], axis=1)
    zeros = jnp.zeros((n_pad, d), jnp.float32)

    sc_scatter = _make_sc_scatter(n_pad, e_pad, d)

    hs = []
    h = x
    for p in params["layers"]:
        parts = sc_scatter(h, ed, zeros)
        h = _tc_layer(h, parts, n_pad, p)
        hs.append(h)

    return _tc_head(hs[0], hs[1], hs[2], batch, params)


# speedup vs baseline: 1.0707x; 1.0168x over previous
import functools

import jax
import jax.numpy as jnp
from jax import lax
from jax.experimental import pallas as pl
from jax.experimental.pallas import tpu as pltpu
from jax.experimental.pallas import tpu_sc as plsc

NC = 2
NS = 16
NW = NC * NS
CH = 128


NBUF = 3


@functools.lru_cache(maxsize=None)
def _make_sc_scatter(n_pad: int, e_pad: int, d: int):
    ew = e_pad // NW
    nch = ew // CH
    rps = n_pad // NS
    mesh = plsc.VectorSubcoreMesh(core_axis_name="c", subcore_axis_name="s")

    @functools.partial(
        pl.kernel,
        out_type=jax.ShapeDtypeStruct((NC * n_pad, d), jnp.float32),
        mesh=mesh,
        scratch_types=(
            [pltpu.VMEM_SHARED((n_pad, d), jnp.float32)]
            + [pltpu.VMEM((2, CH), jnp.int32) for _ in range(NBUF)]
            + [pltpu.VMEM((CH, d), jnp.float32) for _ in range(NBUF)]
            + [pltpu.SemaphoreType.DMA for _ in range(2 * NBUF)]
        ),
    )
    def sc_scatter(h_hbm, ed_hbm, zeros_hbm, out_hbm, acc, *bufs):
        idx = bufs[0:NBUF]
        rows = bufs[NBUF:2 * NBUF]
        gsem = bufs[2 * NBUF:3 * NBUF]
        ssem = bufs[3 * NBUF:4 * NBUF]
        c = lax.axis_index("c")
        s = lax.axis_index("s")
        wid = c * NS + s
        pltpu.sync_copy(zeros_hbm.at[pl.ds(s * rps, rps)],
                        acc.at[pl.ds(s * rps, rps)])
        plsc.subcore_barrier()

        cbase = wid * nch

        def start_gather(g, b):
            pltpu.sync_copy(ed_hbm.at[cbase + g], idx[b])
            pltpu.async_copy(h_hbm.at[idx[b].at[0]], rows[b], gsem[b])

        for b in range(NBUF):
            start_gather(b, b)

        def body(i, carry):
            a = i * NBUF
            for b in range(NBUF):
                pltpu.make_async_copy(h_hbm.at[idx[b].at[0]], rows[b],
                                      gsem[b]).wait()
                pltpu.async_copy(rows[b], acc.at[idx[b].at[1]], ssem[b],
                                 add=True)
            for b in range(NBUF):
                pltpu.make_async_copy(rows[b], acc.at[idx[b].at[1]],
                                      ssem[b]).wait()
                start_gather(a + NBUF + b, b)
            return carry

        lax.fori_loop(0, nch // NBUF, body, 0)
        for b in range(NBUF):
            pltpu.make_async_copy(h_hbm.at[idx[b].at[0]], rows[b],
                                  gsem[b]).wait()

        plsc.subcore_barrier()
        pltpu.sync_copy(acc.at[pl.ds(s * rps, rps)],
                        out_hbm.at[pl.ds(c * n_pad + s * rps, rps)])

    return sc_scatter


def _tc_layer_body(x_ref, parts_ref, w1_ref, b1_ref, w2_ref, b2_ref,
                   g_ref, be_ref, eps_ref, out_ref, *, n_pad: int):
    n = x_ref.shape[0]
    h = ((1.0 + eps_ref[0, 0]) * x_ref[...]
         + parts_ref[0:n] + parts_ref[n_pad:n_pad + n])
    h = jnp.dot(h, w1_ref[...], preferred_element_type=jnp.float32) + b1_ref[...]
    h = jnp.maximum(h, 0.0)
    h = jnp.dot(h, w2_ref[...], preferred_element_type=jnp.float32) + b2_ref[...]
    h = jnp.maximum(h, 0.0)
    mu = jnp.mean(h, axis=0, keepdims=True)
    var = jnp.mean((h - mu) ** 2, axis=0, keepdims=True)
    out_ref[...] = (g_ref[...] * (h - mu) * lax.rsqrt(var + 1e-5)
                    + be_ref[...])


def _tc_layer(x, parts, n_pad, p):
    n, d = x.shape
    h = p["W1"].shape[1]
    return pl.pallas_call(
        functools.partial(_tc_layer_body, n_pad=n_pad),
        out_shape=jax.ShapeDtypeStruct((n, h), jnp.float32),
    )(x, parts, p["W1"], p["b1"].reshape(1, h), p["W2"],
      p["b2"].reshape(1, h), p["gamma"].reshape(1, h),
      p["beta"].reshape(1, h), p["eps"].reshape(1, 1))


def _tc_layer3_head_body(x_ref, parts_ref, w1_ref, b1_ref, w2_ref, b2_ref,
                         g_ref, be_ref, eps_ref, h1_ref, batch_ref,
                         f1w_ref, f1b_ref, f2w_ref, f2b_ref, out_ref,
                         *, n_pad: int, g: int):
    n = x_ref.shape[0]
    h = ((1.0 + eps_ref[0, 0]) * x_ref[...]
         + parts_ref[0:n] + parts_ref[n_pad:n_pad + n])
    h = jnp.dot(h, w1_ref[...], preferred_element_type=jnp.float32) + b1_ref[...]
    h = jnp.maximum(h, 0.0)
    h = jnp.dot(h, w2_ref[...], preferred_element_type=jnp.float32) + b2_ref[...]
    h = jnp.maximum(h, 0.0)
    mu = jnp.mean(h, axis=0, keepdims=True)
    var = jnp.mean((h - mu) ** 2, axis=0, keepdims=True)
    h3 = g_ref[...] * (h - mu) * lax.rsqrt(var + 1e-5) + be_ref[...]
    b = batch_ref[...]
    gid = lax.broadcasted_iota(jnp.int32, (g, b.shape[1]), 0)
    onehot = jnp.where(b == gid, 1.0, 0.0)
    counts = jnp.maximum(jnp.sum(onehot, axis=1, keepdims=True), 1.0)
    s1 = jnp.dot(onehot, h1_ref[...], preferred_element_type=jnp.float32)
    s2 = jnp.dot(onehot, x_ref[...], preferred_element_type=jnp.float32)
    s3 = jnp.dot(onehot, h3, preferred_element_type=jnp.float32)
    pooled = jnp.concatenate([s1, s2, s3], axis=1) / counts
    z = jnp.dot(pooled, f1w_ref[...], preferred_element_type=jnp.float32)
    z = jnp.maximum(z + f1b_ref[...], 0.0)
    logits = jnp.dot(z, f2w_ref[...], preferred_element_type=jnp.float32)
    logits = logits + f2b_ref[...]
    m = jnp.max(logits, axis=1, keepdims=True)
    shifted = logits - m
    out_ref[...] = shifted - jnp.log(
        jnp.sum(jnp.exp(shifted), axis=1, keepdims=True))


def _tc_layer3_head(h2, parts, n_pad, p, h1, batch, params):
    g = 128
    c = params["fc2_W"].shape[1]
    n = h1.shape[0]
    hdim = p["W1"].shape[1]
    fdim = params["fc1_W"].shape[1]
    return pl.pallas_call(
        functools.partial(_tc_layer3_head_body, n_pad=n_pad, g=g),
        out_shape=jax.ShapeDtypeStruct((g, c), jnp.float32),
    )(h2, parts, p["W1"], p["b1"].reshape(1, hdim), p["W2"],
      p["b2"].reshape(1, hdim), p["gamma"].reshape(1, hdim),
      p["beta"].reshape(1, hdim), p["eps"].reshape(1, 1),
      h1, batch.reshape(1, n).astype(jnp.int32),
      params["fc1_W"], params["fc1_b"].reshape(1, fdim),
      params["fc2_W"], params["fc2_b"].reshape(1, c))


def kernel(x, edge_index, batch, params):
    n, d = x.shape
    e = edge_index.shape[1]
    n_pad = ((n + NS * 8 - 1) // (NS * 8)) * NS * 8
    e_pad = ((e + NW * CH * NBUF - 1) // (NW * CH * NBUF)) * NW * CH * NBUF
    src = edge_index[0].astype(jnp.int32)
    dst = edge_index[1].astype(jnp.int32)
    pad = e_pad + NBUF * CH - e
    pad_ar = jnp.arange(pad, dtype=jnp.int32)
    src = jnp.concatenate([src, (pad_ar * 97) % n])
    dst = jnp.concatenate([dst, n + pad_ar % (n_pad - n)])
    ed = jnp.stack([src.reshape(-1, CH), dst.reshape(-1, CH)], axis=1)
    zeros = jnp.zeros((n_pad, d), jnp.float32)

    sc_scatter = _make_sc_scatter(n_pad, e_pad, d)

    layers = params["layers"]
    h1 = _tc_layer(x, sc_scatter(x, ed, zeros), n_pad, layers[0])
    h2 = _tc_layer(h1, sc_scatter(h1, ed, zeros), n_pad, layers[1])
    parts3 = sc_scatter(h2, ed, zeros)
    return _tc_layer3_head(h2, parts3, n_pad, layers[2], h1, batch, params)
